# Initial kernel scaffold; baseline (speedup 1.0000x reference)
import jax
import jax.numpy as jnp
from jax.experimental import pallas as pl


def kernel(x, edge_index, edge_attr, rel_edge_index, rel_edge_type, gW0, gb0, gW1, gb1, gW2, gb2, rW0, rR0, rb0, rW1, rR1, rb1, rW2, rR2, rb2, oW, ob):
    raise NotImplementedError("write your pallas kernel here")


# speedup vs baseline: 19.8271x; 19.8271x over previous
import functools

import jax
import jax.numpy as jnp
from jax import lax
from jax.experimental import pallas as pl
from jax.experimental.pallas import tpu as pltpu
from jax.experimental.pallas import tpu_sc as plsc

NC = 2
NS = 16
NL = 16
NW = NC * NS

CHR = 8
CW = 128
CH = CHR * CW
GH = 512

_f32 = jnp.float32
_i32 = jnp.int32


def _sc_mesh():
    return plsc.VectorSubcoreMesh(core_axis_name="c", subcore_axis_name="s")


_SC_PARAMS = pltpu.CompilerParams(use_tc_tiling_on_sc=False,
                                  needs_layout_passes=False)


def _fill_2d(buf, rows, cols, value):
    def body(i, _):
        for j in range(cols // NL):
            buf[i, pl.ds(j * NL, NL)] = jnp.full((NL,), value, _f32)
        return 0
    lax.fori_loop(0, rows, body, 0, unroll=False)


def _zero_acc_slice(zero_v, acc, row0, nrows, zrows):
    done = 0
    while done < nrows:
        step = min(zrows, nrows - done)
        pltpu.sync_copy(zero_v.at[pl.ds(0, step)],
                        acc.at[pl.ds(row0 + done, step)])
        done += step


def _sc_stats(gdst2, rdst2, rtype, npad, nrpad, r):
    erows = gdst2.shape[0]
    rpt = erows // NS
    deg_rows = npad // NS
    cnt_rows = nrpad // NS

    @functools.partial(
        pl.kernel,
        out_type=(jax.ShapeDtypeStruct((npad, NL), _f32),
                  jax.ShapeDtypeStruct((nrpad, NL), _f32)),
        mesh=_sc_mesh(),
        compiler_params=_SC_PARAMS,
        scratch_types=[
            pltpu.VMEM((CHR, CW), _i32),
            pltpu.VMEM((CH,), _i32),
            pltpu.VMEM((CHR, CW), _i32),
            pltpu.VMEM((CW, NL), _f32),
            pltpu.VMEM_SHARED((npad, NL), _f32),
            pltpu.VMEM_SHARED((nrpad, NL), _f32),
        ],
    )
    def k(gdst_hbm, rdst_hbm, rtype_hbm, deg_out, cnt_out,
          dst_v, typ_v, cidx_v, ones_v, dacc, cacc):
        c = lax.axis_index("c")
        s = lax.axis_index("s")

        _fill_2d(ones_v, CW, NL, 0.0)
        _zero_acc_slice(ones_v, dacc, s * deg_rows, deg_rows, CW)
        _zero_acc_slice(ones_v, cacc, s * cnt_rows, cnt_rows, CW)
        _fill_2d(ones_v, CW, NL, 1.0)
        plsc.subcore_barrier()

        @pl.when(c == 1)
        def _():
            def body(kk, _):
                roff = s * rpt + kk * CHR
                pltpu.sync_copy(gdst_hbm.at[pl.ds(roff, CHR)], dst_v)
                for j in range(CHR):
                    pltpu.sync_copy(ones_v, dacc.at[dst_v.at[j]], add=True)
                return 0
            lax.fori_loop(0, rpt // CHR, body, 0, unroll=False)

        @pl.when(c == 0)
        def _():
            def body(kk, _):
                roff = s * rpt + kk * CHR
                pltpu.sync_copy(rdst_hbm.at[pl.ds(roff, CHR)], dst_v)
                pltpu.sync_copy(rtype_hbm.at[pl.ds(roff * CW, CH)], typ_v)
                for j in range(CHR):
                    for q in range(CW // NL):
                        sl = pl.ds(q * NL, NL)
                        cidx_v[j, sl] = (dst_v[j, sl] * r
                                         + typ_v[pl.ds(j * CW + q * NL, NL)])
                for j in range(CHR):
                    pltpu.sync_copy(ones_v, cacc.at[cidx_v.at[j]], add=True)
                return 0
            lax.fori_loop(0, rpt // CHR, body, 0, unroll=False)

        plsc.subcore_barrier()

        @pl.when(c == 1)
        def _():
            pltpu.sync_copy(dacc.at[pl.ds(s * deg_rows, deg_rows)],
                            deg_out.at[pl.ds(s * deg_rows, deg_rows)])

        @pl.when(c == 0)
        def _():
            pltpu.sync_copy(cacc.at[pl.ds(s * cnt_rows, cnt_rows)],
                            cnt_out.at[pl.ds(s * cnt_rows, cnt_rows)])

    return k(gdst2, rdst2, rtype)


def _sc_edge_prep(rsrc, rdst, rtype, winv_flat, npad, r, erows):
    e = rsrc.shape[0]
    nr = winv_flat.shape[0]
    rpt = erows // NW

    @functools.partial(
        pl.kernel,
        out_type=(jax.ShapeDtypeStruct((erows, CW), _i32),
                  jax.ShapeDtypeStruct((e,), _f32)),
        mesh=_sc_mesh(),
        compiler_params=_SC_PARAMS,
        scratch_types=[
            pltpu.VMEM((CH,), _i32),
            pltpu.VMEM((CH,), _i32),
            pltpu.VMEM((CH,), _i32),
            pltpu.VMEM((CHR, CW), _i32),
            pltpu.VMEM((CH,), _f32),
            pltpu.VMEM((nr,), _f32),
        ],
    )
    def k(src_hbm, dst_hbm, typ_hbm, winv_hbm, rowidx_out, w_out,
          src_v, dst_v, typ_v, idx_v, w_v, winv_v):
        c = lax.axis_index("c")
        s = lax.axis_index("s")
        pltpu.sync_copy(winv_hbm, winv_v)
        rbase = (c * NS + s) * rpt

        def body(kk, _):
            roff = rbase + kk * CHR
            off = roff * CW
            pltpu.sync_copy(src_hbm.at[pl.ds(off, CH)], src_v)
            pltpu.sync_copy(dst_hbm.at[pl.ds(off, CH)], dst_v)
            pltpu.sync_copy(typ_hbm.at[pl.ds(off, CH)], typ_v)

            for j in range(CHR):
                for q in range(CW // NL):
                    sl1 = pl.ds(j * CW + q * NL, NL)
                    t16 = typ_v[sl1]
                    idx_v[j, pl.ds(q * NL, NL)] = t16 * npad + src_v[sl1]
                    w_v[sl1] = plsc.load_gather(winv_v,
                                                [dst_v[sl1] * r + t16])

            pltpu.sync_copy(idx_v, rowidx_out.at[pl.ds(roff, CHR)])
            pltpu.sync_copy(w_v, w_out.at[pl.ds(off, CH)])
            return 0
        lax.fori_loop(0, rpt // CHR, body, 0, unroll=False)

    return k(rsrc, rdst, rtype, winv_flat)


def _sc_gcn_agg(table, src2, dst2, n, h):
    erows = src2.shape[0]
    rpt = erows // NW
    rows_per_tile = n // NS

    @functools.partial(
        pl.kernel,
        out_type=jax.ShapeDtypeStruct((NC, n, h), _f32),
        mesh=_sc_mesh(),
        compiler_params=_SC_PARAMS,
        scratch_types=[
            pltpu.VMEM((CHR, CW), _i32),
            pltpu.VMEM((CHR, CW), _i32),
            pltpu.VMEM((GH, h), _f32),
            pltpu.VMEM_SHARED((n, h), _f32),
            pltpu.SemaphoreType.DMA,
        ],
    )
    def k(table_hbm, src_hbm, dst_hbm, out_hbm,
          idx_v, dst_v, rows_v, acc, sem):
        c = lax.axis_index("c")
        s = lax.axis_index("s")
        _fill_2d(rows_v, GH, h, 0.0)
        _zero_acc_slice(rows_v, acc, s * rows_per_tile, rows_per_tile, GH)
        plsc.subcore_barrier()

        rbase = (c * NS + s) * rpt
        nsub = GH // CW

        def body(kk, _):
            roff = rbase + kk * CHR
            pltpu.sync_copy(src_hbm.at[pl.ds(roff, CHR)], idx_v)
            pltpu.sync_copy(dst_hbm.at[pl.ds(roff, CHR)], dst_v)
            for half in range(CHR // nsub):
                descs = []
                for j in range(nsub):
                    descs.append(pltpu.async_copy(
                        table_hbm.at[idx_v.at[half * nsub + j]],
                        rows_v.at[pl.ds(j * CW, CW)], sem))
                for d in descs:
                    d.wait()
                for j in range(nsub):
                    pltpu.sync_copy(rows_v.at[pl.ds(j * CW, CW)],
                                    acc.at[dst_v.at[half * nsub + j]],
                                    add=True)
            return 0
        lax.fori_loop(0, rpt // CHR, body, 0, unroll=False)

        plsc.subcore_barrier()
        pltpu.sync_copy(acc.at[pl.ds(s * rows_per_tile, rows_per_tile)],
                        out_hbm.at[c, pl.ds(s * rows_per_tile, rows_per_tile)])

    return k(table, src2, dst2)


def _sc_rgcn_agg(table_flat, rowidx2, dst2, w_edge, n, h):
    erows = rowidx2.shape[0]
    rpt = erows // NW
    rows_per_tile = n // NS

    @functools.partial(
        pl.kernel,
        out_type=jax.ShapeDtypeStruct((NC, n, h), _f32),
        mesh=_sc_mesh(),
        compiler_params=_SC_PARAMS,
        scratch_types=[
            pltpu.VMEM((CHR, CW), _i32),
            pltpu.VMEM((CHR, CW), _i32),
            pltpu.VMEM((CH,), _f32),
            pltpu.VMEM((GH, h), _f32),
            pltpu.VMEM_SHARED((n, h), _f32),
            pltpu.SemaphoreType.DMA,
        ],
    )
    def k(table_hbm, rowidx_hbm, dst_hbm, w_hbm, out_hbm,
          idx_v, dst_v, w_v, rows_v, acc, sem):
        c = lax.axis_index("c")
        s = lax.axis_index("s")
        _fill_2d(rows_v, GH, h, 0.0)
        _zero_acc_slice(rows_v, acc, s * rows_per_tile, rows_per_tile, GH)
        plsc.subcore_barrier()

        rbase = (c * NS + s) * rpt
        nsub = GH // CW

        def body(kk, _):
            roff = rbase + kk * CHR
            pltpu.sync_copy(rowidx_hbm.at[pl.ds(roff, CHR)], idx_v)
            pltpu.sync_copy(dst_hbm.at[pl.ds(roff, CHR)], dst_v)
            pltpu.sync_copy(w_hbm.at[pl.ds(roff * CW, CH)], w_v)
            for half in range(CHR // nsub):
                descs = []
                for j in range(nsub):
                    descs.append(pltpu.async_copy(
                        table_hbm.at[idx_v.at[half * nsub + j]],
                        rows_v.at[pl.ds(j * CW, CW)], sem))
                for d in descs:
                    d.wait()

                def sbody(ee, _):
                    we = plsc.load_gather(
                        w_v, [jnp.zeros((NL,), _i32) + half * GH + ee])
                    rsp = jnp.zeros((NL,), _i32) + ee
                    for j in range(h // NL):
                        col = lax.iota(_i32, NL) + j * NL
                        v = plsc.load_gather(rows_v, [rsp, col])
                        plsc.store_scatter(rows_v, [rsp, col], v * we)
                    return 0
                lax.fori_loop(0, GH, sbody, 0, unroll=False)

                for j in range(nsub):
                    pltpu.sync_copy(rows_v.at[pl.ds(j * CW, CW)],
                                    acc.at[dst_v.at[half * nsub + j]],
                                    add=True)
            return 0
        lax.fori_loop(0, rpt // CHR, body, 0, unroll=False)

        plsc.subcore_barrier()
        pltpu.sync_copy(acc.at[pl.ds(s * rows_per_tile, rows_per_tile)],
                        out_hbm.at[c, pl.ds(s * rows_per_tile, rows_per_tile)])

    return k(table_flat, rowidx2, dst2, w_edge)


def _tc_colmap(arr2d, fn, nblk=8):
    n = arr2d.shape[0]
    w = arr2d.shape[1]
    bx = n // nblk

    def f(a_ref, o_ref):
        o_ref[...] = fn(a_ref[:, 0:1])

    return pl.pallas_call(
        f,
        grid=(nblk,),
        in_specs=[pl.BlockSpec((bx, w), lambda i: (i, 0))],
        out_specs=pl.BlockSpec((bx, 1), lambda i: (i, 0)),
        out_shape=jax.ShapeDtypeStruct((n, 1), _f32),
    )(arr2d)


def _tc_table(h, w, dinv):
    n = h.shape[0]
    hd = w.shape[1]

    def f(h_ref, w_ref, d_ref, o_ref):
        o_ref[...] = jnp.dot(h_ref[...], w_ref[...],
                             preferred_element_type=_f32) * d_ref[...]

    return pl.pallas_call(
        f, out_shape=jax.ShapeDtypeStruct((n, hd), _f32))(h, w, dinv)


def _tc_combine_g(accg, table_g, dinv, b2d):
    n, hd = table_g.shape

    def f(a_ref, t_ref, d_ref, b_ref, o_ref):
        tot = a_ref[0] + a_ref[1] + t_ref[...]
        o_ref[...] = jnp.maximum(tot * d_ref[...] + b_ref[...], 0.0)

    return pl.pallas_call(
        f, out_shape=jax.ShapeDtypeStruct((n, hd), _f32))(
            accg, table_g, dinv, b2d)


def _tc_rtables(g, wcat, bcat):
    n, hd = g.shape
    rp1 = wcat.shape[0]

    def f(g_ref, w_ref, b_ref, o_ref):
        o_ref[...] = (jnp.dot(g_ref[...], w_ref[0],
                              preferred_element_type=_f32) + b_ref[0])[None]

    return pl.pallas_call(
        f,
        grid=(rp1,),
        in_specs=[
            pl.BlockSpec((n, hd), lambda i: (0, 0)),
            pl.BlockSpec((1, hd, hd), lambda i: (i, 0, 0)),
            pl.BlockSpec((1, 1, hd), lambda i: (i, 0, 0)),
        ],
        out_specs=pl.BlockSpec((1, n, hd), lambda i: (i, 0, 0)),
        out_shape=jax.ShapeDtypeStruct((rp1, n, hd), _f32),
    )(g, wcat, bcat)


def _tc_next(accr, base):
    n, hd = base.shape

    def f(a_ref, b_ref, o_ref):
        o_ref[...] = jnp.maximum(a_ref[0] + a_ref[1] + b_ref[...], 0.0)

    return pl.pallas_call(
        f, out_shape=jax.ShapeDtypeStruct((n, hd), _f32))(accr, base)


def _tc_final(accr, base, ow, ob2d):
    n = base.shape[0]
    od = ow.shape[1]

    def f(a_ref, b_ref, w_ref, bb_ref, o_ref):
        hh = jnp.maximum(a_ref[0] + a_ref[1] + b_ref[...], 0.0)
        o_ref[...] = jnp.maximum(
            jnp.dot(hh, w_ref[...], preferred_element_type=_f32) + bb_ref[...],
            0.0)

    return pl.pallas_call(
        f, out_shape=jax.ShapeDtypeStruct((n, od), _f32))(accr, base, ow, ob2d)


def kernel(x, edge_index, edge_attr, rel_edge_index, rel_edge_type,
           gW0, gb0, gW1, gb1, gW2, gb2,
           rW0, rR0, rb0, rW1, rR1, rb1, rW2, rR2, rb2,
           oW, ob):
    n, _ = x.shape
    e = edge_index.shape[1]
    hd = gW0.shape[1]
    r = rW0.shape[0]
    npad = -(-n // 128) * 128
    nrpad = npad * r
    epad = -(-e // (NW * CH)) * (NW * CH)
    erows = epad // CW

    gsrc, gdst = edge_index[0], edge_index[1]
    rsrc, rdst = rel_edge_index[0], rel_edge_index[1]

    pad = epad - e
    padrow = (n + (jnp.arange(pad, dtype=_i32) % (npad - n))
              if pad else jnp.zeros((0,), _i32))
    padzero = jnp.zeros((pad,), _i32)

    def p2(a, padv):
        return jnp.concatenate([a, padv]).reshape(erows, CW)

    gsrc2 = p2(gsrc, padrow)
    gdst2 = p2(gdst, padrow)
    rdst2 = p2(rdst, padrow)
    rsrc1 = jnp.concatenate([rsrc, padrow])
    rdst1 = jnp.concatenate([rdst, padrow])
    rtype1 = jnp.concatenate([rel_edge_type, padzero])

    deg2d, cnt2d = _sc_stats(gdst2, rdst2, rtype1, npad, nrpad, r)
    dinv = _tc_colmap(deg2d, lambda d: lax.rsqrt(d + 1.0))
    winv = _tc_colmap(cnt2d, lambda cx: 1.0 / jnp.maximum(cx, 1.0))
    winv_flat = winv.reshape(nrpad)
    rowidx2, w_edge = _sc_edge_prep(rsrc1, rdst1, rtype1, winv_flat,
                                    npad, r, erows)

    gws = [(gW0, gb0), (gW1, gb1), (gW2, gb2)]
    rws = [(rW0, rR0, rb0), (rW1, rR1, rb1), (rW2, rR2, rb2)]

    h = jnp.pad(x, ((0, npad - n), (0, 0)))
    accr = base = None
    for l in range(3):
        gw, gb = gws[l]
        rw, rr, rb = rws[l]
        table_g = _tc_table(h, gw, dinv)
        accg = _sc_gcn_agg(table_g, gsrc2, gdst2, npad, hd)
        g = _tc_combine_g(accg, table_g, dinv, gb.reshape(1, hd))
        wcat = jnp.concatenate([rw, rr[None]], axis=0)
        bcat = jnp.concatenate(
            [jnp.zeros((r, 1, hd), _f32), rb.reshape(1, 1, hd)], axis=0)
        hrcat = _tc_rtables(g, wcat, bcat)
        table_r = hrcat[:r].reshape(r * npad, hd)
        base = hrcat[r]
        accr = _sc_rgcn_agg(table_r, rowidx2, rdst2, w_edge, npad, hd)
        if l < 2:
            h = _tc_next(accr, base)

    return _tc_final(accr, base, oW, ob.reshape(1, oW.shape[1]))[:n]


# speedup vs baseline: 22.9594x; 1.1580x over previous
import functools

import jax
import jax.numpy as jnp
from jax import lax
from jax.experimental import pallas as pl
from jax.experimental.pallas import tpu as pltpu
from jax.experimental.pallas import tpu_sc as plsc

NC = 2
NS = 16
NL = 16
NW = NC * NS

CHR = 8
CW = 128
CH = CHR * CW
GH = 512

_f32 = jnp.float32
_i32 = jnp.int32


def _sc_mesh():
    return plsc.VectorSubcoreMesh(core_axis_name="c", subcore_axis_name="s")


_SC_PARAMS = pltpu.CompilerParams(use_tc_tiling_on_sc=False,
                                  needs_layout_passes=False)


def _fill_2d(buf, rows, cols, value):
    def body(i, _):
        for j in range(cols // NL):
            buf[i, pl.ds(j * NL, NL)] = jnp.full((NL,), value, _f32)
        return 0
    lax.fori_loop(0, rows, body, 0, unroll=False)


def _zero_acc_slice(zero_v, acc, row0, nrows, zrows):
    done = 0
    while done < nrows:
        step = min(zrows, nrows - done)
        pltpu.sync_copy(zero_v.at[pl.ds(0, step)],
                        acc.at[pl.ds(row0 + done, step)])
        done += step


def _sc_stats(gdst2, rdst2, rtype, npad, nrpad, r):
    erows = gdst2.shape[0]
    rpt = erows // NS
    deg_rows = npad // NS
    cnt_rows = nrpad // NS

    @functools.partial(
        pl.kernel,
        out_type=(jax.ShapeDtypeStruct((npad, NL), _f32),
                  jax.ShapeDtypeStruct((nrpad, NL), _f32)),
        mesh=_sc_mesh(),
        compiler_params=_SC_PARAMS,
        scratch_types=[
            pltpu.VMEM((CHR, CW), _i32),
            pltpu.VMEM((CH,), _i32),
            pltpu.VMEM((CHR, CW), _i32),
            pltpu.VMEM((CW, NL), _f32),
            pltpu.VMEM_SHARED((npad, NL), _f32),
            pltpu.VMEM_SHARED((nrpad, NL), _f32),
        ],
    )
    def k(gdst_hbm, rdst_hbm, rtype_hbm, deg_out, cnt_out,
          dst_v, typ_v, cidx_v, ones_v, dacc, cacc):
        c = lax.axis_index("c")
        s = lax.axis_index("s")

        _fill_2d(ones_v, CW, NL, 0.0)
        _zero_acc_slice(ones_v, dacc, s * deg_rows, deg_rows, CW)
        _zero_acc_slice(ones_v, cacc, s * cnt_rows, cnt_rows, CW)
        _fill_2d(ones_v, CW, NL, 1.0)
        plsc.subcore_barrier()

        @pl.when(c == 1)
        def _():
            def body(kk, _):
                roff = s * rpt + kk * CHR
                pltpu.sync_copy(gdst_hbm.at[pl.ds(roff, CHR)], dst_v)
                for j in range(CHR):
                    pltpu.sync_copy(ones_v, dacc.at[dst_v.at[j]], add=True)
                return 0
            lax.fori_loop(0, rpt // CHR, body, 0, unroll=False)

        @pl.when(c == 0)
        def _():
            def body(kk, _):
                roff = s * rpt + kk * CHR
                pltpu.sync_copy(rdst_hbm.at[pl.ds(roff, CHR)], dst_v)
                pltpu.sync_copy(rtype_hbm.at[pl.ds(roff * CW, CH)], typ_v)
                for j in range(CHR):
                    for q in range(CW // NL):
                        sl = pl.ds(q * NL, NL)
                        cidx_v[j, sl] = (dst_v[j, sl] * r
                                         + typ_v[pl.ds(j * CW + q * NL, NL)])
                for j in range(CHR):
                    pltpu.sync_copy(ones_v, cacc.at[cidx_v.at[j]], add=True)
                return 0
            lax.fori_loop(0, rpt // CHR, body, 0, unroll=False)

        plsc.subcore_barrier()

        @pl.when(c == 1)
        def _():
            pltpu.sync_copy(dacc.at[pl.ds(s * deg_rows, deg_rows)],
                            deg_out.at[pl.ds(s * deg_rows, deg_rows)])

        @pl.when(c == 0)
        def _():
            pltpu.sync_copy(cacc.at[pl.ds(s * cnt_rows, cnt_rows)],
                            cnt_out.at[pl.ds(s * cnt_rows, cnt_rows)])

    return k(gdst2, rdst2, rtype)


def _sc_edge_prep(rsrc, rdst, rtype, winv_flat, npad, r, erows):
    e = rsrc.shape[0]
    nr = winv_flat.shape[0]
    rpt = erows // NW

    @functools.partial(
        pl.kernel,
        out_type=(jax.ShapeDtypeStruct((erows, CW), _i32),
                  jax.ShapeDtypeStruct((e,), _f32)),
        mesh=_sc_mesh(),
        compiler_params=_SC_PARAMS,
        scratch_types=[
            pltpu.VMEM((CH,), _i32),
            pltpu.VMEM((CH,), _i32),
            pltpu.VMEM((CH,), _i32),
            pltpu.VMEM((CHR, CW), _i32),
            pltpu.VMEM((CH,), _f32),
            pltpu.VMEM((nr,), _f32),
        ],
    )
    def k(src_hbm, dst_hbm, typ_hbm, winv_hbm, rowidx_out, w_out,
          src_v, dst_v, typ_v, idx_v, w_v, winv_v):
        c = lax.axis_index("c")
        s = lax.axis_index("s")
        pltpu.sync_copy(winv_hbm, winv_v)
        rbase = (c * NS + s) * rpt

        def body(kk, _):
            roff = rbase + kk * CHR
            off = roff * CW
            pltpu.sync_copy(src_hbm.at[pl.ds(off, CH)], src_v)
            pltpu.sync_copy(dst_hbm.at[pl.ds(off, CH)], dst_v)
            pltpu.sync_copy(typ_hbm.at[pl.ds(off, CH)], typ_v)

            for j in range(CHR):
                for q in range(CW // NL):
                    sl1 = pl.ds(j * CW + q * NL, NL)
                    t16 = typ_v[sl1]
                    idx_v[j, pl.ds(q * NL, NL)] = t16 * npad + src_v[sl1]
                    w_v[sl1] = plsc.load_gather(winv_v,
                                                [dst_v[sl1] * r + t16])

            pltpu.sync_copy(idx_v, rowidx_out.at[pl.ds(roff, CHR)])
            pltpu.sync_copy(w_v, w_out.at[pl.ds(off, CH)])
            return 0
        lax.fori_loop(0, rpt // CHR, body, 0, unroll=False)

    return k(rsrc, rdst, rtype, winv_flat)


def _sc_agg(table, src2, dst2, w_edge, n, h):
    scaled = w_edge is not None
    erows = src2.shape[0]
    rpt = erows // NW
    nch = rpt // CHR
    assert nch % 2 == 0
    rows_per_tile = n // NS

    scratch = [
        pltpu.VMEM((CHR, CW), _i32),
        pltpu.VMEM((CHR, CW), _i32),
        pltpu.VMEM((CHR, CW), _i32),
        pltpu.VMEM((CHR, CW), _i32),
        pltpu.VMEM((CW, h), _f32),
        pltpu.VMEM((CW, h), _f32),
        pltpu.VMEM_SHARED((n, h), _f32),
        pltpu.SemaphoreType.DMA,
        pltpu.SemaphoreType.DMA,
        pltpu.SemaphoreType.DMA,
        pltpu.SemaphoreType.DMA,
    ]
    if scaled:
        scratch += [
            pltpu.VMEM((CH,), _f32),
            pltpu.VMEM((CH,), _f32),
            pltpu.SemaphoreType.DMA,
        ]

    @functools.partial(
        pl.kernel,
        out_type=jax.ShapeDtypeStruct((NC, n, h), _f32),
        mesh=_sc_mesh(),
        compiler_params=_SC_PARAMS,
        scratch_types=scratch,
    )
    def k(*refs):
        if scaled:
            (table_hbm, src_hbm, dst_hbm, w_hbm, out_hbm,
             idx0, idx1, dst0, dst1, rb0, rb1, acc,
             sem_ii, sem_id, sem_g0, sem_g1, w0, w1, sem_iw) = refs
            wb = (w0, w1)
        else:
            (table_hbm, src_hbm, dst_hbm, out_hbm,
             idx0, idx1, dst0, dst1, rb0, rb1, acc,
             sem_ii, sem_id, sem_g0, sem_g1) = refs
        c = lax.axis_index("c")
        s = lax.axis_index("s")
        _fill_2d(rb0, CW, h, 0.0)
        _zero_acc_slice(rb0, acc, s * rows_per_tile, rows_per_tile, CW)
        plsc.subcore_barrier()

        rbase = (c * NS + s) * rpt
        idxb = (idx0, idx1)
        dstb = (dst0, dst1)
        rbb = (rb0, rb1)
        semg = (sem_g0, sem_g1)

        def fire_chunk(roff, p):
            pltpu.async_copy(src_hbm.at[pl.ds(roff, CHR)], idxb[p], sem_ii)
            pltpu.async_copy(dst_hbm.at[pl.ds(roff, CHR)], dstb[p], sem_id)
            if scaled:
                pltpu.async_copy(w_hbm.at[pl.ds(roff * CW, CH)], wb[p],
                                 sem_iw)

        def wait_chunk(roff, p):
            pltpu.make_async_copy(src_hbm.at[pl.ds(roff, CHR)], idxb[p],
                                  sem_ii).wait()
            pltpu.make_async_copy(dst_hbm.at[pl.ds(roff, CHR)], dstb[p],
                                  sem_id).wait()
            if scaled:
                pltpu.make_async_copy(w_hbm.at[pl.ds(roff * CW, CH)], wb[p],
                                      sem_iw).wait()

        def scale_rows(rq, wc, j):
            def sb(ee, _):
                we = plsc.load_gather(
                    wc, [jnp.zeros((NL,), _i32) + j * CW + ee])
                rsp = jnp.zeros((NL,), _i32) + ee
                for jj in range(h // NL):
                    col = lax.iota(_i32, NL) + jj * NL
                    v = plsc.load_gather(rq, [rsp, col])
                    plsc.store_scatter(rq, [rsp, col], v * we)
                return 0
            lax.fori_loop(0, CW, sb, 0, unroll=8)

        fire_chunk(rbase, 0)

        def process_chunk(kk2, p):
            idx_c, dst_c = idxb[p], dstb[p]
            roff = rbase + kk2 * CHR
            wait_chunk(roff, p)
            roff_n = rbase + jnp.minimum(kk2 + 1, nch - 1) * CHR
            fire_chunk(roff_n, 1 - p)
            d = pltpu.async_copy(table_hbm.at[idx_c.at[0]], rbb[0], semg[0])
            for j in range(CHR):
                q = j % 2
                dn = None
                if j + 1 < CHR:
                    dn = pltpu.async_copy(table_hbm.at[idx_c.at[j + 1]],
                                          rbb[1 - q], semg[(j + 1) % 2])
                d.wait()
                if scaled:
                    scale_rows(rbb[q], wb[p], j)
                pltpu.sync_copy(rbb[q], acc.at[dst_c.at[j]], add=True)
                d = dn

        def loop_body(kk, _):
            process_chunk(2 * kk, 0)
            process_chunk(2 * kk + 1, 1)
            return 0
        lax.fori_loop(0, nch // 2, loop_body, 0, unroll=False)
        wait_chunk(rbase, nch % 2)

        plsc.subcore_barrier()
        pltpu.sync_copy(acc.at[pl.ds(s * rows_per_tile, rows_per_tile)],
                        out_hbm.at[c, pl.ds(s * rows_per_tile, rows_per_tile)])

    if scaled:
        return k(table, src2, dst2, w_edge)
    return k(table, src2, dst2)


def _tc_colmap(arr2d, fn, nblk=8):
    n = arr2d.shape[0]
    w = arr2d.shape[1]
    bx = n // nblk

    def f(a_ref, o_ref):
        o_ref[...] = fn(a_ref[:, 0:1])

    return pl.pallas_call(
        f,
        grid=(nblk,),
        in_specs=[pl.BlockSpec((bx, w), lambda i: (i, 0))],
        out_specs=pl.BlockSpec((bx, 1), lambda i: (i, 0)),
        out_shape=jax.ShapeDtypeStruct((n, 1), _f32),
    )(arr2d)


def _tc_table(h, w, dinv):
    n = h.shape[0]
    hd = w.shape[1]

    def f(h_ref, w_ref, d_ref, o_ref):
        o_ref[...] = jnp.dot(h_ref[...], w_ref[...],
                             preferred_element_type=_f32) * d_ref[...]

    return pl.pallas_call(
        f, out_shape=jax.ShapeDtypeStruct((n, hd), _f32))(h, w, dinv)


def _tc_combine_g(accg, table_g, dinv, b2d):
    n, hd = table_g.shape

    def f(a_ref, t_ref, d_ref, b_ref, o_ref):
        tot = a_ref[0] + a_ref[1] + t_ref[...]
        o_ref[...] = jnp.maximum(tot * d_ref[...] + b_ref[...], 0.0)

    return pl.pallas_call(
        f, out_shape=jax.ShapeDtypeStruct((n, hd), _f32))(
            accg, table_g, dinv, b2d)


def _tc_rtables(g, wcat, bcat):
    n, hd = g.shape
    rp1 = wcat.shape[0]

    def f(g_ref, w_ref, b_ref, o_ref):
        o_ref[...] = (jnp.dot(g_ref[...], w_ref[0],
                              preferred_element_type=_f32) + b_ref[0])[None]

    return pl.pallas_call(
        f,
        grid=(rp1,),
        in_specs=[
            pl.BlockSpec((n, hd), lambda i: (0, 0)),
            pl.BlockSpec((1, hd, hd), lambda i: (i, 0, 0)),
            pl.BlockSpec((1, 1, hd), lambda i: (i, 0, 0)),
        ],
        out_specs=pl.BlockSpec((1, n, hd), lambda i: (i, 0, 0)),
        out_shape=jax.ShapeDtypeStruct((rp1, n, hd), _f32),
    )(g, wcat, bcat)


def _tc_next(accr, base):
    n, hd = base.shape

    def f(a_ref, b_ref, o_ref):
        o_ref[...] = jnp.maximum(a_ref[0] + a_ref[1] + b_ref[...], 0.0)

    return pl.pallas_call(
        f, out_shape=jax.ShapeDtypeStruct((n, hd), _f32))(accr, base)


def _tc_final(accr, base, ow, ob2d):
    n = base.shape[0]
    od = ow.shape[1]

    def f(a_ref, b_ref, w_ref, bb_ref, o_ref):
        hh = jnp.maximum(a_ref[0] + a_ref[1] + b_ref[...], 0.0)
        o_ref[...] = jnp.maximum(
            jnp.dot(hh, w_ref[...], preferred_element_type=_f32) + bb_ref[...],
            0.0)

    return pl.pallas_call(
        f, out_shape=jax.ShapeDtypeStruct((n, od), _f32))(accr, base, ow, ob2d)


def kernel(x, edge_index, edge_attr, rel_edge_index, rel_edge_type,
           gW0, gb0, gW1, gb1, gW2, gb2,
           rW0, rR0, rb0, rW1, rR1, rb1, rW2, rR2, rb2,
           oW, ob):
    n, _ = x.shape
    e = edge_index.shape[1]
    hd = gW0.shape[1]
    r = rW0.shape[0]
    npad = -(-n // 128) * 128
    nrpad = npad * r
    epad = -(-e // (NW * CH)) * (NW * CH)
    erows = epad // CW

    gsrc, gdst = edge_index[0], edge_index[1]
    rsrc, rdst = rel_edge_index[0], rel_edge_index[1]

    pad = epad - e
    padrow = (n + (jnp.arange(pad, dtype=_i32) % (npad - n))
              if pad else jnp.zeros((0,), _i32))
    padzero = jnp.zeros((pad,), _i32)

    def p2(a, padv):
        return jnp.concatenate([a, padv]).reshape(erows, CW)

    gsrc2 = p2(gsrc, padrow)
    gdst2 = p2(gdst, padrow)
    rdst2 = p2(rdst, padrow)
    rsrc1 = jnp.concatenate([rsrc, padrow])
    rdst1 = jnp.concatenate([rdst, padrow])
    rtype1 = jnp.concatenate([rel_edge_type, padzero])

    deg2d, cnt2d = _sc_stats(gdst2, rdst2, rtype1, npad, nrpad, r)
    dinv = _tc_colmap(deg2d, lambda d: lax.rsqrt(d + 1.0))
    winv = _tc_colmap(cnt2d, lambda cx: 1.0 / jnp.maximum(cx, 1.0))
    winv_flat = winv.reshape(nrpad)
    rowidx2, w_edge = _sc_edge_prep(rsrc1, rdst1, rtype1, winv_flat,
                                    npad, r, erows)

    gws = [(gW0, gb0), (gW1, gb1), (gW2, gb2)]
    rws = [(rW0, rR0, rb0), (rW1, rR1, rb1), (rW2, rR2, rb2)]

    h = jnp.pad(x, ((0, npad - n), (0, 0)))
    accr = base = None
    for l in range(3):
        gw, gb = gws[l]
        rw, rr, rb = rws[l]
        table_g = _tc_table(h, gw, dinv)
        accg = _sc_agg(table_g, gsrc2, gdst2, None, npad, hd)
        g = _tc_combine_g(accg, table_g, dinv, gb.reshape(1, hd))
        wcat = jnp.concatenate([rw, rr[None]], axis=0)
        bcat = jnp.concatenate(
            [jnp.zeros((r, 1, hd), _f32), rb.reshape(1, 1, hd)], axis=0)
        hrcat = _tc_rtables(g, wcat, bcat)
        table_r = hrcat[:r].reshape(r * npad, hd)
        base = hrcat[r]
        accr = _sc_agg(table_r, rowidx2, rdst2, w_edge, npad, hd)
        if l < 2:
            h = _tc_next(accr, base)

    return _tc_final(accr, base, oW, ob.reshape(1, oW.shape[1]))[:n]


# speedup vs baseline: 34.6263x; 1.5082x over previous
import functools

import jax
import jax.numpy as jnp
from jax import lax
from jax.experimental import pallas as pl
from jax.experimental.pallas import tpu as pltpu
from jax.experimental.pallas import tpu_sc as plsc

NC = 2
NS = 16
NL = 16
NW = NC * NS

CHR = 8
CW = 128
CH = CHR * CW
GH = 512

_f32 = jnp.float32
_i32 = jnp.int32


def _sc_mesh():
    return plsc.VectorSubcoreMesh(core_axis_name="c", subcore_axis_name="s")


_SC_PARAMS = pltpu.CompilerParams(use_tc_tiling_on_sc=False,
                                  needs_layout_passes=False)


def _fill_2d(buf, rows, cols, value):
    def body(i, _):
        for j in range(cols // NL):
            buf[i, pl.ds(j * NL, NL)] = jnp.full((NL,), value, _f32)
        return 0
    lax.fori_loop(0, rows, body, 0, unroll=False)


def _zero_acc_slice(zero_v, acc, row0, nrows, zrows):
    done = 0
    while done < nrows:
        step = min(zrows, nrows - done)
        pltpu.sync_copy(zero_v.at[pl.ds(0, step)],
                        acc.at[pl.ds(row0 + done, step)])
        done += step


def _sc_stats(gdst2, rdst2, rtype, npad, nrpad, r):
    erows = gdst2.shape[0]
    rpt = erows // NS
    deg_rows = npad // NS
    cnt_rows = nrpad // NS

    @functools.partial(
        pl.kernel,
        out_type=(jax.ShapeDtypeStruct((npad, NL), _f32),
                  jax.ShapeDtypeStruct((nrpad, NL), _f32)),
        mesh=_sc_mesh(),
        compiler_params=_SC_PARAMS,
        scratch_types=[
            pltpu.VMEM((CHR, CW), _i32),
            pltpu.VMEM((CH,), _i32),
            pltpu.VMEM((CHR, CW), _i32),
            pltpu.VMEM((CW, NL), _f32),
            pltpu.VMEM_SHARED((npad, NL), _f32),
            pltpu.VMEM_SHARED((nrpad, NL), _f32),
        ],
    )
    def k(gdst_hbm, rdst_hbm, rtype_hbm, deg_out, cnt_out,
          dst_v, typ_v, cidx_v, ones_v, dacc, cacc):
        c = lax.axis_index("c")
        s = lax.axis_index("s")

        _fill_2d(ones_v, CW, NL, 0.0)
        _zero_acc_slice(ones_v, dacc, s * deg_rows, deg_rows, CW)
        _zero_acc_slice(ones_v, cacc, s * cnt_rows, cnt_rows, CW)
        _fill_2d(ones_v, CW, NL, 1.0)
        plsc.subcore_barrier()

        @pl.when(c == 1)
        def _():
            def body(kk, _):
                roff = s * rpt + kk * CHR
                pltpu.sync_copy(gdst_hbm.at[pl.ds(roff, CHR)], dst_v)
                for j in range(CHR):
                    pltpu.sync_copy(ones_v, dacc.at[dst_v.at[j]], add=True)
                return 0
            lax.fori_loop(0, rpt // CHR, body, 0, unroll=False)

        @pl.when(c == 0)
        def _():
            def body(kk, _):
                roff = s * rpt + kk * CHR
                pltpu.sync_copy(rdst_hbm.at[pl.ds(roff, CHR)], dst_v)
                pltpu.sync_copy(rtype_hbm.at[pl.ds(roff * CW, CH)], typ_v)
                for j in range(CHR):
                    for q in range(CW // NL):
                        sl = pl.ds(q * NL, NL)
                        cidx_v[j, sl] = (dst_v[j, sl] * r
                                         + typ_v[pl.ds(j * CW + q * NL, NL)])
                for j in range(CHR):
                    pltpu.sync_copy(ones_v, cacc.at[cidx_v.at[j]], add=True)
                return 0
            lax.fori_loop(0, rpt // CHR, body, 0, unroll=False)

        plsc.subcore_barrier()

        @pl.when(c == 1)
        def _():
            pltpu.sync_copy(dacc.at[pl.ds(s * deg_rows, deg_rows)],
                            deg_out.at[pl.ds(s * deg_rows, deg_rows)])

        @pl.when(c == 0)
        def _():
            pltpu.sync_copy(cacc.at[pl.ds(s * cnt_rows, cnt_rows)],
                            cnt_out.at[pl.ds(s * cnt_rows, cnt_rows)])

    return k(gdst2, rdst2, rtype)


def _sc_edge_prep(rsrc, rdst, rtype, winv_flat, npad, r, erows):
    e = rsrc.shape[0]
    nr = winv_flat.shape[0]
    rpt = erows // NW

    @functools.partial(
        pl.kernel,
        out_type=(jax.ShapeDtypeStruct((erows, CW), _i32),
                  jax.ShapeDtypeStruct((e,), _f32)),
        mesh=_sc_mesh(),
        compiler_params=_SC_PARAMS,
        scratch_types=[
            pltpu.VMEM((CH,), _i32),
            pltpu.VMEM((CH,), _i32),
            pltpu.VMEM((CH,), _i32),
            pltpu.VMEM((CHR, CW), _i32),
            pltpu.VMEM((CH,), _f32),
            pltpu.VMEM((nr,), _f32),
        ],
    )
    def k(src_hbm, dst_hbm, typ_hbm, winv_hbm, rowidx_out, w_out,
          src_v, dst_v, typ_v, idx_v, w_v, winv_v):
        c = lax.axis_index("c")
        s = lax.axis_index("s")
        pltpu.sync_copy(winv_hbm, winv_v)
        rbase = (c * NS + s) * rpt

        def body(kk, _):
            roff = rbase + kk * CHR
            off = roff * CW
            pltpu.sync_copy(src_hbm.at[pl.ds(off, CH)], src_v)
            pltpu.sync_copy(dst_hbm.at[pl.ds(off, CH)], dst_v)
            pltpu.sync_copy(typ_hbm.at[pl.ds(off, CH)], typ_v)

            for j in range(CHR):
                for q in range(CW // NL):
                    sl1 = pl.ds(j * CW + q * NL, NL)
                    t16 = typ_v[sl1]
                    idx_v[j, pl.ds(q * NL, NL)] = t16 * npad + src_v[sl1]
                    w_v[sl1] = plsc.load_gather(winv_v,
                                                [dst_v[sl1] * r + t16])

            pltpu.sync_copy(idx_v, rowidx_out.at[pl.ds(roff, CHR)])
            pltpu.sync_copy(w_v, w_out.at[pl.ds(off, CH)])
            return 0
        lax.fori_loop(0, rpt // CHR, body, 0, unroll=False)

    return k(rsrc, rdst, rtype, winv_flat)


def _sc_agg(table, src2, dst2, w_edge, n, h):
    scaled = w_edge is not None
    erows = src2.shape[0]
    rpt = erows // NW
    nch = rpt // CHR
    assert nch % 2 == 0
    rows_per_tile = n // NS

    scratch = [
        pltpu.VMEM((CHR, CW), _i32),
        pltpu.VMEM((CHR, CW), _i32),
        pltpu.VMEM((CHR, CW), _i32),
        pltpu.VMEM((CHR, CW), _i32),
        pltpu.VMEM((CW, h), _f32),
        pltpu.VMEM((CW, h), _f32),
        pltpu.VMEM((CW, h), _f32),
        pltpu.VMEM_SHARED((n, h), _f32),
        pltpu.SemaphoreType.DMA,
        pltpu.SemaphoreType.DMA,
        pltpu.SemaphoreType.DMA,
        pltpu.SemaphoreType.DMA,
    ]
    if scaled:
        scratch += [
            pltpu.VMEM((CH,), _f32),
            pltpu.VMEM((CH,), _f32),
            pltpu.SemaphoreType.DMA,
        ]

    @functools.partial(
        pl.kernel,
        out_type=jax.ShapeDtypeStruct((NC, n, h), _f32),
        mesh=_sc_mesh(),
        compiler_params=_SC_PARAMS,
        scratch_types=scratch,
    )
    def k(*refs):
        if scaled:
            (table_hbm, src_hbm, dst_hbm, w_hbm, out_hbm,
             idx0, idx1, dst0, dst1, rb0, rb1, rout, acc,
             sem_ii, sem_id, sem_g0, sem_g1, w0, w1, sem_iw) = refs
            wb = (w0, w1)
        else:
            (table_hbm, src_hbm, dst_hbm, out_hbm,
             idx0, idx1, dst0, dst1, rb0, rb1, rout, acc,
             sem_ii, sem_id, sem_g0, sem_g1) = refs
        c = lax.axis_index("c")
        s = lax.axis_index("s")
        _fill_2d(rb0, CW, h, 0.0)
        _zero_acc_slice(rb0, acc, s * rows_per_tile, rows_per_tile, CW)
        plsc.subcore_barrier()

        rbase = (c * NS + s) * rpt
        idxb = (idx0, idx1)
        dstb = (dst0, dst1)
        rbb = (rb0, rb1)
        semg = (sem_g0, sem_g1)

        def fire_chunk(roff, p):
            pltpu.async_copy(src_hbm.at[pl.ds(roff, CHR)], idxb[p], sem_ii)
            pltpu.async_copy(dst_hbm.at[pl.ds(roff, CHR)], dstb[p], sem_id)
            if scaled:
                pltpu.async_copy(w_hbm.at[pl.ds(roff * CW, CH)], wb[p],
                                 sem_iw)

        def wait_chunk(roff, p):
            pltpu.make_async_copy(src_hbm.at[pl.ds(roff, CHR)], idxb[p],
                                  sem_ii).wait()
            pltpu.make_async_copy(dst_hbm.at[pl.ds(roff, CHR)], dstb[p],
                                  sem_id).wait()
            if scaled:
                pltpu.make_async_copy(w_hbm.at[pl.ds(roff * CW, CH)], wb[p],
                                      sem_iw).wait()

        def scale_rows(rin, wc, j):
            @plsc.parallel_loop(0, CW, 1, unroll=8)
            def _(ee):
                we = plsc.load_gather(
                    wc, [jnp.zeros((NL,), _i32) + j * CW + ee])
                rsp = jnp.zeros((NL,), _i32) + ee
                for jj in range(h // NL):
                    col = lax.iota(_i32, NL) + jj * NL
                    v = plsc.load_gather(rin, [rsp, col])
                    plsc.store_scatter(rout, [rsp, col], v * we)

        fire_chunk(rbase, 0)

        def process_chunk(kk2, p):
            idx_c, dst_c = idxb[p], dstb[p]
            roff = rbase + kk2 * CHR
            wait_chunk(roff, p)
            roff_n = rbase + jnp.minimum(kk2 + 1, nch - 1) * CHR
            fire_chunk(roff_n, 1 - p)
            d = pltpu.async_copy(table_hbm.at[idx_c.at[0]], rbb[0], semg[0])
            for j in range(CHR):
                q = j % 2
                dn = None
                if j + 1 < CHR:
                    dn = pltpu.async_copy(table_hbm.at[idx_c.at[j + 1]],
                                          rbb[1 - q], semg[(j + 1) % 2])
                d.wait()
                if scaled:
                    scale_rows(rbb[q], wb[p], j)
                    pltpu.sync_copy(rout, acc.at[dst_c.at[j]], add=True)
                else:
                    pltpu.sync_copy(rbb[q], acc.at[dst_c.at[j]], add=True)
                d = dn

        def loop_body(kk, _):
            process_chunk(2 * kk, 0)
            process_chunk(2 * kk + 1, 1)
            return 0
        lax.fori_loop(0, nch // 2, loop_body, 0, unroll=False)
        wait_chunk(rbase, nch % 2)

        plsc.subcore_barrier()
        pltpu.sync_copy(acc.at[pl.ds(s * rows_per_tile, rows_per_tile)],
                        out_hbm.at[c, pl.ds(s * rows_per_tile, rows_per_tile)])

    if scaled:
        return k(table, src2, dst2, w_edge)
    return k(table, src2, dst2)


def _tc_colmap(arr2d, fn, nblk=8):
    n = arr2d.shape[0]
    w = arr2d.shape[1]
    bx = n // nblk

    def f(a_ref, o_ref):
        o_ref[...] = fn(a_ref[:, 0:1])

    return pl.pallas_call(
        f,
        grid=(nblk,),
        in_specs=[pl.BlockSpec((bx, w), lambda i: (i, 0))],
        out_specs=pl.BlockSpec((bx, 1), lambda i: (i, 0)),
        out_shape=jax.ShapeDtypeStruct((n, 1), _f32),
    )(arr2d)


def _tc_table(h, w, dinv):
    n = h.shape[0]
    hd = w.shape[1]

    def f(h_ref, w_ref, d_ref, o_ref):
        o_ref[...] = jnp.dot(h_ref[...], w_ref[...],
                             preferred_element_type=_f32) * d_ref[...]

    return pl.pallas_call(
        f, out_shape=jax.ShapeDtypeStruct((n, hd), _f32))(h, w, dinv)


def _tc_combine_g(accg, table_g, dinv, b2d):
    n, hd = table_g.shape

    def f(a_ref, t_ref, d_ref, b_ref, o_ref):
        tot = a_ref[0] + a_ref[1] + t_ref[...]
        o_ref[...] = jnp.maximum(tot * d_ref[...] + b_ref[...], 0.0)

    return pl.pallas_call(
        f, out_shape=jax.ShapeDtypeStruct((n, hd), _f32))(
            accg, table_g, dinv, b2d)


def _tc_rtables(g, wcat, bcat):
    n, hd = g.shape
    rp1 = wcat.shape[0]

    def f(g_ref, w_ref, b_ref, o_ref):
        o_ref[...] = (jnp.dot(g_ref[...], w_ref[0],
                              preferred_element_type=_f32) + b_ref[0])[None]

    return pl.pallas_call(
        f,
        grid=(rp1,),
        in_specs=[
            pl.BlockSpec((n, hd), lambda i: (0, 0)),
            pl.BlockSpec((1, hd, hd), lambda i: (i, 0, 0)),
            pl.BlockSpec((1, 1, hd), lambda i: (i, 0, 0)),
        ],
        out_specs=pl.BlockSpec((1, n, hd), lambda i: (i, 0, 0)),
        out_shape=jax.ShapeDtypeStruct((rp1, n, hd), _f32),
    )(g, wcat, bcat)


def _tc_next(accr, base):
    n, hd = base.shape

    def f(a_ref, b_ref, o_ref):
        o_ref[...] = jnp.maximum(a_ref[0] + a_ref[1] + b_ref[...], 0.0)

    return pl.pallas_call(
        f, out_shape=jax.ShapeDtypeStruct((n, hd), _f32))(accr, base)


def _tc_final(accr, base, ow, ob2d):
    n = base.shape[0]
    od = ow.shape[1]

    def f(a_ref, b_ref, w_ref, bb_ref, o_ref):
        hh = jnp.maximum(a_ref[0] + a_ref[1] + b_ref[...], 0.0)
        o_ref[...] = jnp.maximum(
            jnp.dot(hh, w_ref[...], preferred_element_type=_f32) + bb_ref[...],
            0.0)

    return pl.pallas_call(
        f, out_shape=jax.ShapeDtypeStruct((n, od), _f32))(accr, base, ow, ob2d)


def kernel(x, edge_index, edge_attr, rel_edge_index, rel_edge_type,
           gW0, gb0, gW1, gb1, gW2, gb2,
           rW0, rR0, rb0, rW1, rR1, rb1, rW2, rR2, rb2,
           oW, ob):
    n, _ = x.shape
    e = edge_index.shape[1]
    hd = gW0.shape[1]
    r = rW0.shape[0]
    npad = -(-n // 128) * 128
    nrpad = npad * r
    epad = -(-e // (NW * CH)) * (NW * CH)
    erows = epad // CW

    gsrc, gdst = edge_index[0], edge_index[1]
    rsrc, rdst = rel_edge_index[0], rel_edge_index[1]

    pad = epad - e
    padrow = (n + (jnp.arange(pad, dtype=_i32) % (npad - n))
              if pad else jnp.zeros((0,), _i32))
    padzero = jnp.zeros((pad,), _i32)

    def p2(a, padv):
        return jnp.concatenate([a, padv]).reshape(erows, CW)

    gsrc2 = p2(gsrc, padrow)
    gdst2 = p2(gdst, padrow)
    rdst2 = p2(rdst, padrow)
    rsrc1 = jnp.concatenate([rsrc, padrow])
    rdst1 = jnp.concatenate([rdst, padrow])
    rtype1 = jnp.concatenate([rel_edge_type, padzero])

    deg2d, cnt2d = _sc_stats(gdst2, rdst2, rtype1, npad, nrpad, r)
    dinv = _tc_colmap(deg2d, lambda d: lax.rsqrt(d + 1.0))
    winv = _tc_colmap(cnt2d, lambda cx: 1.0 / jnp.maximum(cx, 1.0))
    winv_flat = winv.reshape(nrpad)
    rowidx2, w_edge = _sc_edge_prep(rsrc1, rdst1, rtype1, winv_flat,
                                    npad, r, erows)

    gws = [(gW0, gb0), (gW1, gb1), (gW2, gb2)]
    rws = [(rW0, rR0, rb0), (rW1, rR1, rb1), (rW2, rR2, rb2)]

    h = jnp.pad(x, ((0, npad - n), (0, 0)))
    accr = base = None
    for l in range(3):
        gw, gb = gws[l]
        rw, rr, rb = rws[l]
        table_g = _tc_table(h, gw, dinv)
        accg = _sc_agg(table_g, gsrc2, gdst2, None, npad, hd)
        g = _tc_combine_g(accg, table_g, dinv, gb.reshape(1, hd))
        wcat = jnp.concatenate([rw, rr[None]], axis=0)
        bcat = jnp.concatenate(
            [jnp.zeros((r, 1, hd), _f32), rb.reshape(1, 1, hd)], axis=0)
        hrcat = _tc_rtables(g, wcat, bcat)
        table_r = hrcat[:r].reshape(r * npad, hd)
        base = hrcat[r]
        accr = _sc_agg(table_r, rowidx2, rdst2, w_edge, npad, hd)
        if l < 2:
            h = _tc_next(accr, base)

    return _tc_final(accr, base, oW, ob.reshape(1, oW.shape[1]))[:n]


# speedup vs baseline: 36.2307x; 1.0463x over previous
import functools

import jax
import jax.numpy as jnp
from jax import lax
from jax.experimental import pallas as pl
from jax.experimental.pallas import tpu as pltpu
from jax.experimental.pallas import tpu_sc as plsc

NC = 2
NS = 16
NL = 16
NW = NC * NS

CHR = 8
CW = 128
CH = CHR * CW
GH = 512

_f32 = jnp.float32
_i32 = jnp.int32


def _sc_mesh():
    return plsc.VectorSubcoreMesh(core_axis_name="c", subcore_axis_name="s")


_SC_PARAMS = pltpu.CompilerParams(use_tc_tiling_on_sc=False,
                                  needs_layout_passes=False)


def _fill_2d(buf, rows, cols, value):
    def body(i, _):
        for j in range(cols // NL):
            buf[i, pl.ds(j * NL, NL)] = jnp.full((NL,), value, _f32)
        return 0
    lax.fori_loop(0, rows, body, 0, unroll=False)


def _zero_acc_slice(zero_v, acc, row0, nrows, zrows):
    done = 0
    while done < nrows:
        step = min(zrows, nrows - done)
        pltpu.sync_copy(zero_v.at[pl.ds(0, step)],
                        acc.at[pl.ds(row0 + done, step)])
        done += step


def _sc_stats(gdst2, rdst2, rtype, npad, nrpad, r):
    erows = gdst2.shape[0]
    rpt = erows // NS
    deg_rows = npad // NS
    cnt_rows = nrpad // NS

    @functools.partial(
        pl.kernel,
        out_type=(jax.ShapeDtypeStruct((npad, NL), _f32),
                  jax.ShapeDtypeStruct((nrpad, NL), _f32)),
        mesh=_sc_mesh(),
        compiler_params=_SC_PARAMS,
        scratch_types=[
            pltpu.VMEM((CHR, CW), _i32),
            pltpu.VMEM((CH,), _i32),
            pltpu.VMEM((CHR, CW), _i32),
            pltpu.VMEM((CW, NL), _f32),
            pltpu.VMEM_SHARED((npad, NL), _f32),
            pltpu.VMEM_SHARED((nrpad, NL), _f32),
            pltpu.SemaphoreType.DMA,
        ],
    )
    def k(gdst_hbm, rdst_hbm, rtype_hbm, deg_out, cnt_out,
          dst_v, typ_v, cidx_v, ones_v, dacc, cacc, sem_s):
        c = lax.axis_index("c")
        s = lax.axis_index("s")

        _fill_2d(ones_v, CW, NL, 0.0)
        _zero_acc_slice(ones_v, dacc, s * deg_rows, deg_rows, CW)
        _zero_acc_slice(ones_v, cacc, s * cnt_rows, cnt_rows, CW)
        _fill_2d(ones_v, CW, NL, 1.0)
        plsc.subcore_barrier()

        @pl.when(c == 1)
        def _():
            def body(kk, _):
                roff = s * rpt + kk * CHR
                pltpu.sync_copy(gdst_hbm.at[pl.ds(roff, CHR)], dst_v)
                descs = [pltpu.async_copy(ones_v, dacc.at[dst_v.at[j]],
                                          sem_s, add=True)
                         for j in range(CHR)]
                for d in descs:
                    d.wait()
                return 0
            lax.fori_loop(0, rpt // CHR, body, 0, unroll=False)

        @pl.when(c == 0)
        def _():
            def body(kk, _):
                roff = s * rpt + kk * CHR
                pltpu.sync_copy(rdst_hbm.at[pl.ds(roff, CHR)], dst_v)
                pltpu.sync_copy(rtype_hbm.at[pl.ds(roff * CW, CH)], typ_v)
                for j in range(CHR):
                    for q in range(CW // NL):
                        sl = pl.ds(q * NL, NL)
                        cidx_v[j, sl] = (dst_v[j, sl] * r
                                         + typ_v[pl.ds(j * CW + q * NL, NL)])
                descs = [pltpu.async_copy(ones_v, cacc.at[cidx_v.at[j]],
                                          sem_s, add=True)
                         for j in range(CHR)]
                for d in descs:
                    d.wait()
                return 0
            lax.fori_loop(0, rpt // CHR, body, 0, unroll=False)

        plsc.subcore_barrier()

        @pl.when(c == 1)
        def _():
            pltpu.sync_copy(dacc.at[pl.ds(s * deg_rows, deg_rows)],
                            deg_out.at[pl.ds(s * deg_rows, deg_rows)])

        @pl.when(c == 0)
        def _():
            pltpu.sync_copy(cacc.at[pl.ds(s * cnt_rows, cnt_rows)],
                            cnt_out.at[pl.ds(s * cnt_rows, cnt_rows)])

    return k(gdst2, rdst2, rtype)


def _sc_edge_prep(rsrc, rdst, rtype, winv_flat, npad, r, erows):
    e = rsrc.shape[0]
    nr = winv_flat.shape[0]
    rpt = erows // NW

    @functools.partial(
        pl.kernel,
        out_type=(jax.ShapeDtypeStruct((erows, CW), _i32),
                  jax.ShapeDtypeStruct((e,), _f32)),
        mesh=_sc_mesh(),
        compiler_params=_SC_PARAMS,
        scratch_types=[
            pltpu.VMEM((CH,), _i32),
            pltpu.VMEM((CH,), _i32),
            pltpu.VMEM((CH,), _i32),
            pltpu.VMEM((CHR, CW), _i32),
            pltpu.VMEM((CH,), _f32),
            pltpu.VMEM((nr,), _f32),
        ],
    )
    def k(src_hbm, dst_hbm, typ_hbm, winv_hbm, rowidx_out, w_out,
          src_v, dst_v, typ_v, idx_v, w_v, winv_v):
        c = lax.axis_index("c")
        s = lax.axis_index("s")
        pltpu.sync_copy(winv_hbm, winv_v)
        rbase = (c * NS + s) * rpt

        def body(kk, _):
            roff = rbase + kk * CHR
            off = roff * CW
            pltpu.sync_copy(src_hbm.at[pl.ds(off, CH)], src_v)
            pltpu.sync_copy(dst_hbm.at[pl.ds(off, CH)], dst_v)
            pltpu.sync_copy(typ_hbm.at[pl.ds(off, CH)], typ_v)

            for j in range(CHR):
                for q in range(CW // NL):
                    sl1 = pl.ds(j * CW + q * NL, NL)
                    t16 = typ_v[sl1]
                    idx_v[j, pl.ds(q * NL, NL)] = t16 * npad + src_v[sl1]
                    w_v[sl1] = plsc.load_gather(winv_v,
                                                [dst_v[sl1] * r + t16])

            pltpu.sync_copy(idx_v, rowidx_out.at[pl.ds(roff, CHR)])
            pltpu.sync_copy(w_v, w_out.at[pl.ds(off, CH)])
            return 0
        lax.fori_loop(0, rpt // CHR, body, 0, unroll=False)

    return k(rsrc, rdst, rtype, winv_flat)


def _sc_agg(table, src2, dst2, w_edge, n, h):
    scaled = w_edge is not None
    erows = src2.shape[0]
    rpt = erows // NW
    nch = rpt // CHR
    assert nch % 2 == 0
    rows_per_tile = n // NS

    scratch = [
        pltpu.VMEM((CHR, CW), _i32),
        pltpu.VMEM((CHR, CW), _i32),
        pltpu.VMEM((CHR, CW), _i32),
        pltpu.VMEM((CHR, CW), _i32),
        pltpu.VMEM((CW, h), _f32),
        pltpu.VMEM((CW, h), _f32),
        pltpu.VMEM((CW, h), _f32),
        pltpu.VMEM((CW, h), _f32),
        pltpu.VMEM_SHARED((n, h), _f32),
        pltpu.SemaphoreType.DMA,
        pltpu.SemaphoreType.DMA,
        pltpu.SemaphoreType.DMA,
        pltpu.SemaphoreType.DMA,
        pltpu.SemaphoreType.DMA,
        pltpu.SemaphoreType.DMA,
    ]
    if scaled:
        scratch += [
            pltpu.VMEM((CH,), _f32),
            pltpu.VMEM((CH,), _f32),
            pltpu.SemaphoreType.DMA,
        ]

    @functools.partial(
        pl.kernel,
        out_type=jax.ShapeDtypeStruct((NC, n, h), _f32),
        mesh=_sc_mesh(),
        compiler_params=_SC_PARAMS,
        scratch_types=scratch,
    )
    def k(*refs):
        if scaled:
            (table_hbm, src_hbm, dst_hbm, w_hbm, out_hbm,
             idx0, idx1, dst0, dst1, rb0, rb1, rout0, rout1, acc,
             sem_ii, sem_id, sem_g0, sem_g1, sem_s0, sem_s1,
             w0, w1, sem_iw) = refs
            wb = (w0, w1)
        else:
            (table_hbm, src_hbm, dst_hbm, out_hbm,
             idx0, idx1, dst0, dst1, rb0, rb1, rout0, rout1, acc,
             sem_ii, sem_id, sem_g0, sem_g1, sem_s0, sem_s1) = refs
        c = lax.axis_index("c")
        s = lax.axis_index("s")
        _fill_2d(rb0, CW, h, 0.0)
        _zero_acc_slice(rb0, acc, s * rows_per_tile, rows_per_tile, CW)
        plsc.subcore_barrier()

        rbase = (c * NS + s) * rpt
        idxb = (idx0, idx1)
        dstb = (dst0, dst1)
        rbb = (rb0, rb1)
        semg = (sem_g0, sem_g1)

        def fire_chunk(roff, p):
            pltpu.async_copy(src_hbm.at[pl.ds(roff, CHR)], idxb[p], sem_ii)
            pltpu.async_copy(dst_hbm.at[pl.ds(roff, CHR)], dstb[p], sem_id)
            if scaled:
                pltpu.async_copy(w_hbm.at[pl.ds(roff * CW, CH)], wb[p],
                                 sem_iw)

        def wait_chunk(roff, p):
            pltpu.make_async_copy(src_hbm.at[pl.ds(roff, CHR)], idxb[p],
                                  sem_ii).wait()
            pltpu.make_async_copy(dst_hbm.at[pl.ds(roff, CHR)], dstb[p],
                                  sem_id).wait()
            if scaled:
                pltpu.make_async_copy(w_hbm.at[pl.ds(roff * CW, CH)], wb[p],
                                      sem_iw).wait()

        routb = (rout0, rout1)
        sems = (sem_s0, sem_s1)

        def scale_rows(rin, rdst_buf, wc, j):
            @plsc.parallel_loop(0, CW, 1, unroll=8)
            def _(ee):
                we = plsc.load_gather(
                    wc, [jnp.zeros((NL,), _i32) + j * CW + ee])
                rsp = jnp.zeros((NL,), _i32) + ee
                for jj in range(h // NL):
                    col = lax.iota(_i32, NL) + jj * NL
                    v = plsc.load_gather(rin, [rsp, col])
                    plsc.store_scatter(rdst_buf, [rsp, col], v * we)

        fire_chunk(rbase, 0)

        def process_chunk(kk2, p):
            idx_c, dst_c = idxb[p], dstb[p]
            roff = rbase + kk2 * CHR
            wait_chunk(roff, p)
            roff_n = rbase + jnp.minimum(kk2 + 1, nch - 1) * CHR
            fire_chunk(roff_n, 1 - p)
            d = pltpu.async_copy(table_hbm.at[idx_c.at[0]], rbb[0], semg[0])
            sdescs = [None] * CHR
            for j in range(CHR):
                q = j % 2
                dn = None
                if j + 1 < CHR:
                    if not scaled and sdescs[j - 1] is not None:
                        sdescs[j - 1].wait()
                        sdescs[j - 1] = None
                    dn = pltpu.async_copy(table_hbm.at[idx_c.at[j + 1]],
                                          rbb[1 - q], semg[(j + 1) % 2])
                d.wait()
                if scaled:
                    if sdescs[j - 2] is not None:
                        sdescs[j - 2].wait()
                        sdescs[j - 2] = None
                    scale_rows(rbb[q], routb[q], wb[p], j)
                    sdescs[j] = pltpu.async_copy(
                        routb[q], acc.at[dst_c.at[j]], sems[q], add=True)
                else:
                    sdescs[j] = pltpu.async_copy(
                        rbb[q], acc.at[dst_c.at[j]], sems[q], add=True)
                d = dn
            for d in sdescs:
                if d is not None:
                    d.wait()

        def loop_body(kk, _):
            process_chunk(2 * kk, 0)
            process_chunk(2 * kk + 1, 1)
            return 0
        lax.fori_loop(0, nch // 2, loop_body, 0, unroll=False)
        wait_chunk(rbase, nch % 2)

        plsc.subcore_barrier()
        pltpu.sync_copy(acc.at[pl.ds(s * rows_per_tile, rows_per_tile)],
                        out_hbm.at[c, pl.ds(s * rows_per_tile, rows_per_tile)])

    if scaled:
        return k(table, src2, dst2, w_edge)
    return k(table, src2, dst2)


def _tc_colmap(arr2d, fn, nblk=8):
    n = arr2d.shape[0]
    w = arr2d.shape[1]
    bx = n // nblk

    def f(a_ref, o_ref):
        o_ref[...] = fn(a_ref[:, 0:1])

    return pl.pallas_call(
        f,
        grid=(nblk,),
        in_specs=[pl.BlockSpec((bx, w), lambda i: (i, 0))],
        out_specs=pl.BlockSpec((bx, 1), lambda i: (i, 0)),
        out_shape=jax.ShapeDtypeStruct((n, 1), _f32),
    )(arr2d)


def _tc_table(h, w, dinv):
    n = h.shape[0]
    hd = w.shape[1]

    def f(h_ref, w_ref, d_ref, o_ref):
        o_ref[...] = jnp.dot(h_ref[...], w_ref[...],
                             preferred_element_type=_f32) * d_ref[...]

    return pl.pallas_call(
        f, out_shape=jax.ShapeDtypeStruct((n, hd), _f32))(h, w, dinv)


def _tc_combine_g(accg, table_g, dinv, b2d):
    n, hd = table_g.shape

    def f(a_ref, t_ref, d_ref, b_ref, o_ref):
        tot = a_ref[0] + a_ref[1] + t_ref[...]
        o_ref[...] = jnp.maximum(tot * d_ref[...] + b_ref[...], 0.0)

    return pl.pallas_call(
        f, out_shape=jax.ShapeDtypeStruct((n, hd), _f32))(
            accg, table_g, dinv, b2d)


def _tc_rtables(g, wcat, bcat):
    n, hd = g.shape
    rp1 = wcat.shape[0]

    def f(g_ref, w_ref, b_ref, o_ref):
        o_ref[...] = (jnp.dot(g_ref[...], w_ref[0],
                              preferred_element_type=_f32) + b_ref[0])[None]

    return pl.pallas_call(
        f,
        grid=(rp1,),
        in_specs=[
            pl.BlockSpec((n, hd), lambda i: (0, 0)),
            pl.BlockSpec((1, hd, hd), lambda i: (i, 0, 0)),
            pl.BlockSpec((1, 1, hd), lambda i: (i, 0, 0)),
        ],
        out_specs=pl.BlockSpec((1, n, hd), lambda i: (i, 0, 0)),
        out_shape=jax.ShapeDtypeStruct((rp1, n, hd), _f32),
    )(g, wcat, bcat)


def _tc_next(accr, base):
    n, hd = base.shape

    def f(a_ref, b_ref, o_ref):
        o_ref[...] = jnp.maximum(a_ref[0] + a_ref[1] + b_ref[...], 0.0)

    return pl.pallas_call(
        f, out_shape=jax.ShapeDtypeStruct((n, hd), _f32))(accr, base)


def _tc_final(accr, base, ow, ob2d):
    n = base.shape[0]
    od = ow.shape[1]

    def f(a_ref, b_ref, w_ref, bb_ref, o_ref):
        hh = jnp.maximum(a_ref[0] + a_ref[1] + b_ref[...], 0.0)
        o_ref[...] = jnp.maximum(
            jnp.dot(hh, w_ref[...], preferred_element_type=_f32) + bb_ref[...],
            0.0)

    return pl.pallas_call(
        f, out_shape=jax.ShapeDtypeStruct((n, od), _f32))(accr, base, ow, ob2d)


def kernel(x, edge_index, edge_attr, rel_edge_index, rel_edge_type,
           gW0, gb0, gW1, gb1, gW2, gb2,
           rW0, rR0, rb0, rW1, rR1, rb1, rW2, rR2, rb2,
           oW, ob):
    n, _ = x.shape
    e = edge_index.shape[1]
    hd = gW0.shape[1]
    r = rW0.shape[0]
    npad = -(-n // 128) * 128
    nrpad = npad * r
    epad = -(-e // (NW * CH)) * (NW * CH)
    erows = epad // CW

    gsrc, gdst = edge_index[0], edge_index[1]
    rsrc, rdst = rel_edge_index[0], rel_edge_index[1]

    pad = epad - e
    padrow = (n + (jnp.arange(pad, dtype=_i32) % (npad - n))
              if pad else jnp.zeros((0,), _i32))
    padzero = jnp.zeros((pad,), _i32)

    def p2(a, padv):
        return jnp.concatenate([a, padv]).reshape(erows, CW)

    gsrc2 = p2(gsrc, padrow)
    gdst2 = p2(gdst, padrow)
    rdst2 = p2(rdst, padrow)
    rsrc1 = jnp.concatenate([rsrc, padrow])
    rdst1 = jnp.concatenate([rdst, padrow])
    rtype1 = jnp.concatenate([rel_edge_type, padzero])

    deg2d, cnt2d = _sc_stats(gdst2, rdst2, rtype1, npad, nrpad, r)
    dinv = _tc_colmap(deg2d, lambda d: lax.rsqrt(d + 1.0))
    winv = _tc_colmap(cnt2d, lambda cx: 1.0 / jnp.maximum(cx, 1.0))
    winv_flat = winv.reshape(nrpad)
    rowidx2, w_edge = _sc_edge_prep(rsrc1, rdst1, rtype1, winv_flat,
                                    npad, r, erows)

    gws = [(gW0, gb0), (gW1, gb1), (gW2, gb2)]
    rws = [(rW0, rR0, rb0), (rW1, rR1, rb1), (rW2, rR2, rb2)]

    h = jnp.pad(x, ((0, npad - n), (0, 0)))
    accr = base = None
    for l in range(3):
        gw, gb = gws[l]
        rw, rr, rb = rws[l]
        table_g = _tc_table(h, gw, dinv)
        accg = _sc_agg(table_g, gsrc2, gdst2, None, npad, hd)
        g = _tc_combine_g(accg, table_g, dinv, gb.reshape(1, hd))
        wcat = jnp.concatenate([rw, rr[None]], axis=0)
        bcat = jnp.concatenate(
            [jnp.zeros((r, 1, hd), _f32), rb.reshape(1, 1, hd)], axis=0)
        hrcat = _tc_rtables(g, wcat, bcat)
        table_r = hrcat[:r].reshape(r * npad, hd)
        base = hrcat[r]
        accr = _sc_agg(table_r, rowidx2, rdst2, w_edge, npad, hd)
        if l < 2:
            h = _tc_next(accr, base)

    return _tc_final(accr, base, oW, ob.reshape(1, oW.shape[1]))[:n]


# speedup vs baseline: 37.3082x; 1.0297x over previous
import functools

import jax
import jax.numpy as jnp
from jax import lax
from jax.experimental import pallas as pl
from jax.experimental.pallas import tpu as pltpu
from jax.experimental.pallas import tpu_sc as plsc

NC = 2
NS = 16
NL = 16
NW = NC * NS

CHR = 8
CW = 128
CH = CHR * CW
GH = 512

_f32 = jnp.float32
_i32 = jnp.int32


def _sc_mesh():
    return plsc.VectorSubcoreMesh(core_axis_name="c", subcore_axis_name="s")


_SC_PARAMS = pltpu.CompilerParams(use_tc_tiling_on_sc=False,
                                  needs_layout_passes=False)


def _fill_2d(buf, rows, cols, value):
    def body(i, _):
        for j in range(cols // NL):
            buf[i, pl.ds(j * NL, NL)] = jnp.full((NL,), value, _f32)
        return 0
    lax.fori_loop(0, rows, body, 0, unroll=False)


def _zero_acc_slice(zero_v, acc, row0, nrows, zrows):
    done = 0
    while done < nrows:
        step = min(zrows, nrows - done)
        pltpu.sync_copy(zero_v.at[pl.ds(0, step)],
                        acc.at[pl.ds(row0 + done, step)])
        done += step


def _sc_stats(gdst2, rdst2, rtype, npad, nrpad, r):
    erows = gdst2.shape[0]
    rpt = erows // NS
    deg_rows = npad // NS
    cnt_rows = nrpad // NS

    @functools.partial(
        pl.kernel,
        out_type=(jax.ShapeDtypeStruct((npad, NL), _f32),
                  jax.ShapeDtypeStruct((nrpad, NL), _f32)),
        mesh=_sc_mesh(),
        compiler_params=_SC_PARAMS,
        scratch_types=[
            pltpu.VMEM((CHR, CW), _i32),
            pltpu.VMEM((CH,), _i32),
            pltpu.VMEM((CHR, CW), _i32),
            pltpu.VMEM((CW, NL), _f32),
            pltpu.VMEM_SHARED((npad, NL), _f32),
            pltpu.VMEM_SHARED((nrpad, NL), _f32),
            pltpu.SemaphoreType.DMA,
        ],
    )
    def k(gdst_hbm, rdst_hbm, rtype_hbm, deg_out, cnt_out,
          dst_v, typ_v, cidx_v, ones_v, dacc, cacc, sem_s):
        c = lax.axis_index("c")
        s = lax.axis_index("s")

        _fill_2d(ones_v, CW, NL, 0.0)
        _zero_acc_slice(ones_v, dacc, s * deg_rows, deg_rows, CW)
        _zero_acc_slice(ones_v, cacc, s * cnt_rows, cnt_rows, CW)
        _fill_2d(ones_v, CW, NL, 1.0)
        plsc.subcore_barrier()

        @pl.when(c == 1)
        def _():
            def body(kk, _):
                roff = s * rpt + kk * CHR
                pltpu.sync_copy(gdst_hbm.at[pl.ds(roff, CHR)], dst_v)
                descs = [pltpu.async_copy(ones_v, dacc.at[dst_v.at[j]],
                                          sem_s, add=True)
                         for j in range(CHR)]
                for d in descs:
                    d.wait()
                return 0
            lax.fori_loop(0, rpt // CHR, body, 0, unroll=False)

        @pl.when(c == 0)
        def _():
            def body(kk, _):
                roff = s * rpt + kk * CHR
                pltpu.sync_copy(rdst_hbm.at[pl.ds(roff, CHR)], dst_v)
                pltpu.sync_copy(rtype_hbm.at[pl.ds(roff * CW, CH)], typ_v)
                for j in range(CHR):
                    for q in range(CW // NL):
                        sl = pl.ds(q * NL, NL)
                        cidx_v[j, sl] = (dst_v[j, sl] * r
                                         + typ_v[pl.ds(j * CW + q * NL, NL)])
                descs = [pltpu.async_copy(ones_v, cacc.at[cidx_v.at[j]],
                                          sem_s, add=True)
                         for j in range(CHR)]
                for d in descs:
                    d.wait()
                return 0
            lax.fori_loop(0, rpt // CHR, body, 0, unroll=False)

        plsc.subcore_barrier()

        @pl.when(c == 1)
        def _():
            pltpu.sync_copy(dacc.at[pl.ds(s * deg_rows, deg_rows)],
                            deg_out.at[pl.ds(s * deg_rows, deg_rows)])

        @pl.when(c == 0)
        def _():
            pltpu.sync_copy(cacc.at[pl.ds(s * cnt_rows, cnt_rows)],
                            cnt_out.at[pl.ds(s * cnt_rows, cnt_rows)])

    return k(gdst2, rdst2, rtype)


def _sc_edge_prep(rsrc, rdst, rtype, winv_flat, npad, r, erows):
    e = rsrc.shape[0]
    nr = winv_flat.shape[0]
    rpt = erows // NW

    @functools.partial(
        pl.kernel,
        out_type=(jax.ShapeDtypeStruct((erows, CW), _i32),
                  jax.ShapeDtypeStruct((e,), _f32)),
        mesh=_sc_mesh(),
        compiler_params=_SC_PARAMS,
        scratch_types=[
            pltpu.VMEM((CH,), _i32),
            pltpu.VMEM((CH,), _i32),
            pltpu.VMEM((CH,), _i32),
            pltpu.VMEM((CHR, CW), _i32),
            pltpu.VMEM((CH,), _f32),
            pltpu.VMEM((nr,), _f32),
        ],
    )
    def k(src_hbm, dst_hbm, typ_hbm, winv_hbm, rowidx_out, w_out,
          src_v, dst_v, typ_v, idx_v, w_v, winv_v):
        c = lax.axis_index("c")
        s = lax.axis_index("s")
        pltpu.sync_copy(winv_hbm, winv_v)
        rbase = (c * NS + s) * rpt

        def body(kk, _):
            roff = rbase + kk * CHR
            off = roff * CW
            pltpu.sync_copy(src_hbm.at[pl.ds(off, CH)], src_v)
            pltpu.sync_copy(dst_hbm.at[pl.ds(off, CH)], dst_v)
            pltpu.sync_copy(typ_hbm.at[pl.ds(off, CH)], typ_v)

            for j in range(CHR):
                for q in range(CW // NL):
                    sl1 = pl.ds(j * CW + q * NL, NL)
                    t16 = typ_v[sl1]
                    idx_v[j, pl.ds(q * NL, NL)] = t16 * npad + src_v[sl1]
                    w_v[sl1] = plsc.load_gather(winv_v,
                                                [dst_v[sl1] * r + t16])

            pltpu.sync_copy(idx_v, rowidx_out.at[pl.ds(roff, CHR)])
            pltpu.sync_copy(w_v, w_out.at[pl.ds(off, CH)])
            return 0
        lax.fori_loop(0, rpt // CHR, body, 0, unroll=False)

    return k(rsrc, rdst, rtype, winv_flat)


def _sc_agg(table, src2, dst2, w_edge, n, h):
    scaled = w_edge is not None
    erows = src2.shape[0]
    rpt = erows // NW
    nch = rpt // CHR
    assert nch % 2 == 0
    rows_per_tile = n // NS

    scratch = [
        pltpu.VMEM((CHR, CW), _i32),
        pltpu.VMEM((CHR, CW), _i32),
        pltpu.VMEM((CHR, CW), _i32),
        pltpu.VMEM((CHR, CW), _i32),
        pltpu.VMEM((CW, h), _f32),
        pltpu.VMEM((CW, h), _f32),
        pltpu.VMEM((CW, h), _f32),
        pltpu.VMEM((CW, h), _f32),
        pltpu.VMEM_SHARED((n, h), _f32),
        pltpu.SemaphoreType.DMA,
        pltpu.SemaphoreType.DMA,
        pltpu.SemaphoreType.DMA,
        pltpu.SemaphoreType.DMA,
        pltpu.SemaphoreType.DMA,
        pltpu.SemaphoreType.DMA,
    ]
    if scaled:
        scratch += [
            pltpu.VMEM((CH,), _f32),
            pltpu.VMEM((CH,), _f32),
            pltpu.SemaphoreType.DMA,
        ]

    @functools.partial(
        pl.kernel,
        out_type=jax.ShapeDtypeStruct((NC, n, h), _f32),
        mesh=_sc_mesh(),
        compiler_params=_SC_PARAMS,
        scratch_types=scratch,
    )
    def k(*refs):
        if scaled:
            (table_hbm, src_hbm, dst_hbm, w_hbm, out_hbm,
             idx0, idx1, dst0, dst1, rb0, rb1, rout0, rout1, acc,
             sem_ii, sem_id, sem_g0, sem_g1, sem_s0, sem_s1,
             w0, w1, sem_iw) = refs
            wb = (w0, w1)
        else:
            (table_hbm, src_hbm, dst_hbm, out_hbm,
             idx0, idx1, dst0, dst1, rb0, rb1, rout0, rout1, acc,
             sem_ii, sem_id, sem_g0, sem_g1, sem_s0, sem_s1) = refs
        c = lax.axis_index("c")
        s = lax.axis_index("s")
        _fill_2d(rb0, CW, h, 0.0)
        _zero_acc_slice(rb0, acc, s * rows_per_tile, rows_per_tile, CW)
        plsc.subcore_barrier()

        rbase = (c * NS + s) * rpt
        idxb = (idx0, idx1)
        dstb = (dst0, dst1)
        rbb = (rb0, rb1)
        semg = (sem_g0, sem_g1)

        def fire_chunk(roff, p):
            pltpu.async_copy(src_hbm.at[pl.ds(roff, CHR)], idxb[p], sem_ii)
            pltpu.async_copy(dst_hbm.at[pl.ds(roff, CHR)], dstb[p], sem_id)
            if scaled:
                pltpu.async_copy(w_hbm.at[pl.ds(roff * CW, CH)], wb[p],
                                 sem_iw)

        def wait_chunk(roff, p):
            pltpu.make_async_copy(src_hbm.at[pl.ds(roff, CHR)], idxb[p],
                                  sem_ii).wait()
            pltpu.make_async_copy(dst_hbm.at[pl.ds(roff, CHR)], dstb[p],
                                  sem_id).wait()
            if scaled:
                pltpu.make_async_copy(w_hbm.at[pl.ds(roff * CW, CH)], wb[p],
                                      sem_iw).wait()

        routb = (rout0, rout1)
        sems = (sem_s0, sem_s1)

        def scale_rows(rin, rdst_buf, wc, j):
            @plsc.parallel_loop(0, CW, 1, unroll=8)
            def _(ee):
                we = plsc.load_gather(
                    wc, [jnp.zeros((NL,), _i32) + j * CW + ee])
                rsp = jnp.zeros((NL,), _i32) + ee
                for jj in range(h // NL):
                    col = lax.iota(_i32, NL) + jj * NL
                    v = plsc.load_gather(rin, [rsp, col])
                    plsc.store_scatter(rdst_buf, [rsp, col], v * we)

        fire_chunk(rbase, 0)

        def process_chunk(kk2, p):
            idx_c, dst_c = idxb[p], dstb[p]
            roff = rbase + kk2 * CHR
            wait_chunk(roff, p)
            roff_n = rbase + jnp.minimum(kk2 + 1, nch - 1) * CHR
            fire_chunk(roff_n, 1 - p)
            d = pltpu.async_copy(table_hbm.at[idx_c.at[0]], rbb[0], semg[0])
            sdescs = [None] * CHR
            for j in range(CHR):
                q = j % 2
                dn = None
                if j + 1 < CHR:
                    if not scaled and sdescs[j - 1] is not None:
                        sdescs[j - 1].wait()
                        sdescs[j - 1] = None
                    dn = pltpu.async_copy(table_hbm.at[idx_c.at[j + 1]],
                                          rbb[1 - q], semg[(j + 1) % 2])
                d.wait()
                if scaled:
                    if sdescs[j - 2] is not None:
                        sdescs[j - 2].wait()
                        sdescs[j - 2] = None
                    scale_rows(rbb[q], routb[q], wb[p], j)
                    sdescs[j] = pltpu.async_copy(
                        routb[q], acc.at[dst_c.at[j]], sems[q], add=True)
                else:
                    sdescs[j] = pltpu.async_copy(
                        rbb[q], acc.at[dst_c.at[j]], sems[q], add=True)
                d = dn
            for d in sdescs:
                if d is not None:
                    d.wait()

        def loop_body(kk, _):
            process_chunk(2 * kk, 0)
            process_chunk(2 * kk + 1, 1)
            return 0
        lax.fori_loop(0, nch // 2, loop_body, 0, unroll=False)
        wait_chunk(rbase, nch % 2)

        plsc.subcore_barrier()
        pltpu.sync_copy(acc.at[pl.ds(s * rows_per_tile, rows_per_tile)],
                        out_hbm.at[c, pl.ds(s * rows_per_tile, rows_per_tile)])

    if scaled:
        return k(table, src2, dst2, w_edge)
    return k(table, src2, dst2)


def _tc_colmap(arr2d, fn, nblk=8):
    n = arr2d.shape[0]
    w = arr2d.shape[1]
    bx = n // nblk

    def f(a_ref, o_ref):
        o_ref[...] = fn(a_ref[:, 0:1])

    return pl.pallas_call(
        f,
        grid=(nblk,),
        in_specs=[pl.BlockSpec((bx, w), lambda i: (i, 0))],
        out_specs=pl.BlockSpec((bx, 1), lambda i: (i, 0)),
        out_shape=jax.ShapeDtypeStruct((n, 1), _f32),
    )(arr2d)


def _tc_table(h, w, dinv):
    n = h.shape[0]
    hd = w.shape[1]

    def f(h_ref, w_ref, d_ref, o_ref):
        o_ref[...] = jnp.dot(h_ref[...], w_ref[...],
                             preferred_element_type=_f32) * d_ref[...]

    return pl.pallas_call(
        f, out_shape=jax.ShapeDtypeStruct((n, hd), _f32))(h, w, dinv)


def _tc_combine_rtables(accg, table_g, dinv, b2d, wcat, bcat):
    n, hd = table_g.shape
    rp1 = wcat.shape[0]

    def f(a_ref, t_ref, d_ref, b_ref, w_ref, bc_ref, o_ref, g_scr):
        @pl.when(pl.program_id(0) == 0)
        def _():
            tot = a_ref[0] + a_ref[1] + t_ref[...]
            g_scr[...] = jnp.maximum(tot * d_ref[...] + b_ref[...], 0.0)
        o_ref[...] = (jnp.dot(g_scr[...], w_ref[0],
                              preferred_element_type=_f32) + bc_ref[0])[None]

    return pl.pallas_call(
        f,
        grid=(rp1,),
        in_specs=[
            pl.BlockSpec((2, n, hd), lambda i: (0, 0, 0)),
            pl.BlockSpec((n, hd), lambda i: (0, 0)),
            pl.BlockSpec((n, 1), lambda i: (0, 0)),
            pl.BlockSpec((1, hd), lambda i: (0, 0)),
            pl.BlockSpec((1, hd, hd), lambda i: (i, 0, 0)),
            pl.BlockSpec((1, 1, hd), lambda i: (i, 0, 0)),
        ],
        out_specs=pl.BlockSpec((1, n, hd), lambda i: (i, 0, 0)),
        out_shape=jax.ShapeDtypeStruct((rp1, n, hd), _f32),
        scratch_shapes=[pltpu.VMEM((n, hd), _f32)],
    )(accg, table_g, dinv, b2d, wcat, bcat)


def _tc_table_next(accr, base, w, dinv):
    n, hd = base.shape

    def f(a_ref, b_ref, w_ref, d_ref, o_ref):
        hh = jnp.maximum(a_ref[0] + a_ref[1] + b_ref[...], 0.0)
        o_ref[...] = jnp.dot(hh, w_ref[...],
                             preferred_element_type=_f32) * d_ref[...]

    return pl.pallas_call(
        f, out_shape=jax.ShapeDtypeStruct((n, w.shape[1]), _f32))(
            accr, base, w, dinv)


def _tc_final(accr, base, ow, ob2d):
    n = base.shape[0]
    od = ow.shape[1]

    def f(a_ref, b_ref, w_ref, bb_ref, o_ref):
        hh = jnp.maximum(a_ref[0] + a_ref[1] + b_ref[...], 0.0)
        o_ref[...] = jnp.maximum(
            jnp.dot(hh, w_ref[...], preferred_element_type=_f32) + bb_ref[...],
            0.0)

    return pl.pallas_call(
        f, out_shape=jax.ShapeDtypeStruct((n, od), _f32))(accr, base, ow, ob2d)


def kernel(x, edge_index, edge_attr, rel_edge_index, rel_edge_type,
           gW0, gb0, gW1, gb1, gW2, gb2,
           rW0, rR0, rb0, rW1, rR1, rb1, rW2, rR2, rb2,
           oW, ob):
    n, _ = x.shape
    e = edge_index.shape[1]
    hd = gW0.shape[1]
    r = rW0.shape[0]
    npad = -(-n // 128) * 128
    nrpad = npad * r
    epad = -(-e // (NW * CH)) * (NW * CH)
    erows = epad // CW

    gsrc, gdst = edge_index[0], edge_index[1]
    rsrc, rdst = rel_edge_index[0], rel_edge_index[1]

    pad = epad - e
    padrow = (n + (jnp.arange(pad, dtype=_i32) % (npad - n))
              if pad else jnp.zeros((0,), _i32))
    padzero = jnp.zeros((pad,), _i32)

    def p2(a, padv):
        return jnp.concatenate([a, padv]).reshape(erows, CW)

    gsrc2 = p2(gsrc, padrow)
    gdst2 = p2(gdst, padrow)
    rdst2 = p2(rdst, padrow)
    rsrc1 = jnp.concatenate([rsrc, padrow])
    rdst1 = jnp.concatenate([rdst, padrow])
    rtype1 = jnp.concatenate([rel_edge_type, padzero])

    deg2d, cnt2d = _sc_stats(gdst2, rdst2, rtype1, npad, nrpad, r)
    dinv = _tc_colmap(deg2d, lambda d: lax.rsqrt(d + 1.0))
    winv = _tc_colmap(cnt2d, lambda cx: 1.0 / jnp.maximum(cx, 1.0))
    winv_flat = winv.reshape(nrpad)
    rowidx2, w_edge = _sc_edge_prep(rsrc1, rdst1, rtype1, winv_flat,
                                    npad, r, erows)

    gws = [(gW0, gb0), (gW1, gb1), (gW2, gb2)]
    rws = [(rW0, rR0, rb0), (rW1, rR1, rb1), (rW2, rR2, rb2)]

    xp = jnp.pad(x, ((0, npad - n), (0, 0)))
    table_g = _tc_table(xp, gws[0][0], dinv)
    accr = base = None
    for l in range(3):
        gw, gb = gws[l]
        rw, rr, rb = rws[l]
        accg = _sc_agg(table_g, gsrc2, gdst2, None, npad, hd)
        wcat = jnp.concatenate([rw, rr[None]], axis=0)
        bcat = jnp.concatenate(
            [jnp.zeros((r, 1, hd), _f32), rb.reshape(1, 1, hd)], axis=0)
        hrcat = _tc_combine_rtables(accg, table_g, dinv, gb.reshape(1, hd),
                                    wcat, bcat)
        table_r = hrcat[:r].reshape(r * npad, hd)
        base = hrcat[r]
        accr = _sc_agg(table_r, rowidx2, rdst2, w_edge, npad, hd)
        if l < 2:
            table_g = _tc_table_next(accr, base, gws[l + 1][0], dinv)

    return _tc_final(accr, base, oW, ob.reshape(1, oW.shape[1]))[:n]
